# baseline (speedup 1.0000x reference)
import jax
import jax.numpy as jnp
from jax import lax
from jax.experimental import layout as jlayout
from jax.experimental import pallas as pl
from jax.experimental.pallas import tpu as pltpu
from jax.experimental.pallas import tpu_sc as plsc

B = 16384
D = 32
PACK = 128 // D
NB = 1000000 // PACK
NC = 2
NS = 16
L = 16
NW = NC * NS
BPW = B // NW
CH = 128
NCH = BPW // CH
NG = CH // L


def _fpmc_body(uid_hbm, lic_hbm, nit_hbm, ui_hbm, iu_hbm, li_hbm, il_hbm,
               out_hbm, idx_u, idx_l, idx_n, q_u, q_l, q_n,
               g_ui, g_iu, g_li, g_il, out_v, sem):
    wid = lax.axis_index("s") * NC + lax.axis_index("c")
    base = wid * BPW

    pltpu.sync_copy(uid_hbm.at[pl.ds(base, BPW)], idx_u)
    pltpu.sync_copy(lic_hbm.at[pl.ds(base, BPW)], idx_l)
    pltpu.sync_copy(nit_hbm.at[pl.ds(base, BPW)], idx_n)

    def shift(i, carry):
        s = i * L
        idx = pl.ds(s, L)
        q_u[idx] = lax.shift_right_logical(idx_u[idx], 2)
        q_l[idx] = lax.shift_right_logical(idx_l[idx], 2)
        q_n[idx] = lax.shift_right_logical(idx_n[idx], 2)
        return carry

    lax.fori_loop(0, BPW // L, shift, 0)

    lanes = lax.iota(jnp.int32, L)

    def batch(c, carry):
        b0 = c * CH
        copies = [
            pltpu.async_copy(ui_hbm.at[q_u.at[pl.ds(b0, CH)]], g_ui, sem),
            pltpu.async_copy(iu_hbm.at[q_n.at[pl.ds(b0, CH)]], g_iu, sem),
            pltpu.async_copy(li_hbm.at[q_l.at[pl.ds(b0, CH)]], g_li, sem),
            pltpu.async_copy(il_hbm.at[q_n.at[pl.ds(b0, CH)]], g_il, sem),
        ]
        for cp in copies:
            cp.wait()

        def group(g, carry2):
            rows = g * L + lanes
            off_u = (idx_u[pl.ds(b0 + g * L, L)] & (PACK - 1)) * D
            off_l = (idx_l[pl.ds(b0 + g * L, L)] & (PACK - 1)) * D
            off_n = (idx_n[pl.ds(b0 + g * L, L)] & (PACK - 1)) * D
            acc = jnp.zeros((L,), jnp.float32)
            for d in range(D):
                acc = acc + (plsc.load_gather(g_ui, [rows, off_u + d]) *
                             plsc.load_gather(g_iu, [rows, off_n + d]))
                acc = acc + (plsc.load_gather(g_li, [rows, off_l + d]) *
                             plsc.load_gather(g_il, [rows, off_n + d]))
            out_v[pl.ds(b0 + g * L, L)] = 1.0 / (1.0 + jnp.exp(-acc))
            return carry2

        lax.fori_loop(0, NG, group, 0)
        return carry

    lax.fori_loop(0, NCH, batch, 0)
    pltpu.sync_copy(out_v, out_hbm.at[pl.ds(base, BPW)])


@jax.jit
def _fpmc(uid, lic, nit, UI, IU, LI, IL):
    fn = pl.kernel(
        _fpmc_body,
        out_type=jax.ShapeDtypeStruct((B,), jnp.float32),
        mesh=plsc.VectorSubcoreMesh(core_axis_name="c", subcore_axis_name="s",
                                    num_cores=NC, num_subcores=NS),
        scratch_types=[
            pltpu.VMEM((BPW,), jnp.int32),
            pltpu.VMEM((BPW,), jnp.int32),
            pltpu.VMEM((BPW,), jnp.int32),
            pltpu.VMEM((BPW,), jnp.int32),
            pltpu.VMEM((BPW,), jnp.int32),
            pltpu.VMEM((BPW,), jnp.int32),
            pltpu.VMEM((CH, 128), jnp.float32),
            pltpu.VMEM((CH, 128), jnp.float32),
            pltpu.VMEM((CH, 128), jnp.float32),
            pltpu.VMEM((CH, 128), jnp.float32),
            pltpu.VMEM((BPW,), jnp.float32),
            pltpu.SemaphoreType.DMA,
        ],
        compiler_params=pltpu.CompilerParams(use_tc_tiling_on_sc=True,
                                             needs_layout_passes=False),
    )
    return fn(uid, lic, nit, UI, IU, LI, IL)


def kernel(user_id, item_last_click, next_item, UI, IU, LI, IL):
    uid = user_id.reshape(-1).astype(jnp.int32)
    lic = item_last_click.reshape(-1).astype(jnp.int32)
    nit = next_item.reshape(-1).astype(jnp.int32)
    fmt = jlayout.Layout(major_to_minor=(0, 1), tiling=((8, 128),))
    t = [jlayout.with_layout_constraint(x.reshape(NB, 128), fmt)
         for x in (UI, IU, LI, IL)]
    return _fpmc(uid, lic, nit, *t)


# speedup vs baseline: 3.6076x; 3.6076x over previous
import jax
import jax.numpy as jnp
from jax import lax
from jax.experimental import pallas as pl
from jax.experimental.pallas import tpu as pltpu
from jax.experimental.pallas import tpu_sc as plsc

B = 16384
D = 32
NC = 2
NS = 16
L = 16
NW = NC * NS
BPW = B // NW
NG = BPW // L
K = 2
WPG = L // K
BLK = 128
SLOT = 4 * K * BLK


def _fpmc_body(uid_hbm, lic_hbm, nit_hbm, ui_hbm, iu_hbm, li_hbm, il_hbm,
               out_hbm, idx_u, idx_l, idx_n, blk0, blk1, out_v, sem0, sem1):
    wid = lax.axis_index("s") * NC + lax.axis_index("c")
    base = wid * BPW

    pltpu.sync_copy(uid_hbm.at[pl.ds(base, BPW)], idx_u)
    pltpu.sync_copy(lic_hbm.at[pl.ds(base, BPW)], idx_l)
    pltpu.sync_copy(nit_hbm.at[pl.ds(base, BPW)], idx_n)

    lanes = lax.iota(jnp.int32, L)
    tabs = (ui_hbm, iu_hbm, li_hbm, il_hbm)
    bufs = (blk0, blk1)
    sems = (sem0, sem1)

    def load_vecs(g):
        s = pl.ds(g * L, L)
        return idx_u[s], idx_l[s], idx_n[s]

    def fire(vecs, w2, pb):
        uvec, lvec, nvec = vecs
        buf, sem = bufs[pb], sems[pb]
        for k in range(K):
            lane = w2 * K + k
            vs = (uvec[lane], nvec[lane], lvec[lane], nvec[lane])
            for t in range(4):
                j = pl.multiple_of((vs[t] >> 7) * BLK, BLK)
                pltpu.async_copy(tabs[t].at[:, pl.ds(j, BLK)],
                                 buf.at[:, pl.ds((k * 4 + t) * BLK, BLK)],
                                 sem)

    def drain(pb):
        buf, sem = bufs[pb], sems[pb]
        for i in range(4 * K):
            pltpu.make_async_copy(ui_hbm.at[:, pl.ds(0, BLK)],
                                  buf.at[:, pl.ds(i * BLK, BLK)], sem).wait()

    def compute(vecs, w2, pb, score):
        uvec, lvec, nvec = vecs
        buf = bufs[pb]
        for k in range(K):
            lane = w2 * K + k
            vs = (uvec[lane], nvec[lane], lvec[lane], nvec[lane])
            cols = []
            for t in range(4):
                c = jnp.broadcast_to(vs[t] & (BLK - 1), (L,))
                col = c + (k * 4 + t) * BLK
                lo = plsc.load_gather(buf, [lanes, col])
                hi = plsc.load_gather(buf, [lanes + L, col])
                cols.append((lo, hi))
            p = (cols[0][0] * cols[1][0] + cols[0][1] * cols[1][1] +
                 cols[2][0] * cols[3][0] + cols[2][1] * cols[3][1])
            s = jnp.sum(p)
            score = jnp.where(lanes == lane, s, score)
        return score

    def group_body(g, carry):
        vecs = load_vecs(g)
        score = jnp.zeros((L,), jnp.float32)
        for w2 in range(WPG):
            if w2 + 1 < WPG:
                fire(vecs, w2 + 1, (w2 + 1) % 2)
            else:
                @pl.when(g + 1 < NG)
                def _():
                    fire(load_vecs(g + 1), 0, 0)
            drain(w2 % 2)
            score = compute(vecs, w2, w2 % 2, score)
        out_v[pl.ds(g * L, L)] = 1.0 / (1.0 + jnp.exp(-score))
        return carry

    fire(load_vecs(0), 0, 0)
    lax.fori_loop(0, NG, group_body, 0)
    pltpu.sync_copy(out_v, out_hbm.at[pl.ds(base, BPW)])


@jax.jit
def _fpmc(uid, lic, nit, UIt, IUt, LIt, ILt):
    fn = pl.kernel(
        _fpmc_body,
        out_type=jax.ShapeDtypeStruct((B,), jnp.float32),
        mesh=plsc.VectorSubcoreMesh(core_axis_name="c", subcore_axis_name="s",
                                    num_cores=NC, num_subcores=NS),
        scratch_types=[
            pltpu.VMEM((BPW,), jnp.int32),
            pltpu.VMEM((BPW,), jnp.int32),
            pltpu.VMEM((BPW,), jnp.int32),
            pltpu.VMEM((D, SLOT), jnp.float32),
            pltpu.VMEM((D, SLOT), jnp.float32),
            pltpu.VMEM((BPW,), jnp.float32),
            pltpu.SemaphoreType.DMA,
            pltpu.SemaphoreType.DMA,
        ],
        compiler_params=pltpu.CompilerParams(use_tc_tiling_on_sc=True,
                                             needs_layout_passes=False),
    )
    return fn(uid, lic, nit, UIt, IUt, LIt, ILt)


def kernel(user_id, item_last_click, next_item, UI, IU, LI, IL):
    uid = user_id.reshape(-1).astype(jnp.int32)
    lic = item_last_click.reshape(-1).astype(jnp.int32)
    nit = next_item.reshape(-1).astype(jnp.int32)
    return _fpmc(uid, lic, nit, UI.T, IU.T, LI.T, IL.T)


# speedup vs baseline: 4.0130x; 1.1124x over previous
import jax
import jax.numpy as jnp
from jax import lax
from jax.experimental import pallas as pl
from jax.experimental.pallas import tpu as pltpu
from jax.experimental.pallas import tpu_sc as plsc

B = 16384
D = 32
NC = 2
NS = 16
L = 16
NW = NC * NS
BPW = B // NW
NG = BPW // L
K = 1
WPG = L // K
NSET = 4
BLK = 128
SLOT = 4 * K * BLK


def _fpmc_body(uid_hbm, lic_hbm, nit_hbm, ui_hbm, iu_hbm, li_hbm, il_hbm,
               out_hbm, idx_u, idx_l, idx_n, blk0, blk1, blk2, blk3, out_v,
               sem0, sem1, sem2, sem3):
    wid = lax.axis_index("s") * NC + lax.axis_index("c")
    base = wid * BPW

    pltpu.sync_copy(uid_hbm.at[pl.ds(base, BPW)], idx_u)
    pltpu.sync_copy(lic_hbm.at[pl.ds(base, BPW)], idx_l)
    pltpu.sync_copy(nit_hbm.at[pl.ds(base, BPW)], idx_n)

    lanes = lax.iota(jnp.int32, L)
    tabs = (ui_hbm, iu_hbm, li_hbm, il_hbm)
    bufs = (blk0, blk1, blk2, blk3)
    sems = (sem0, sem1, sem2, sem3)

    def load_vecs(g):
        s = pl.ds(g * L, L)
        return idx_u[s], idx_l[s], idx_n[s]

    def fire(vecs, w2, pb):
        uvec, lvec, nvec = vecs
        buf, sem = bufs[pb], sems[pb]
        for k in range(K):
            lane = w2 * K + k
            vs = (uvec[lane], nvec[lane], lvec[lane], nvec[lane])
            for t in range(4):
                j = pl.multiple_of((vs[t] >> 7) * BLK, BLK)
                pltpu.async_copy(tabs[t].at[:, pl.ds(j, BLK)],
                                 buf.at[:, pl.ds((k * 4 + t) * BLK, BLK)],
                                 sem)

    def drain(pb):
        buf, sem = bufs[pb], sems[pb]
        for i in range(4 * K):
            pltpu.make_async_copy(ui_hbm.at[:, pl.ds(0, BLK)],
                                  buf.at[:, pl.ds(i * BLK, BLK)], sem).wait()

    def compute(vecs, w2, pb, score):
        uvec, lvec, nvec = vecs
        buf = bufs[pb]
        for k in range(K):
            lane = w2 * K + k
            vs = (uvec[lane], nvec[lane], lvec[lane], nvec[lane])
            cols = []
            for t in range(4):
                c = jnp.broadcast_to(vs[t] & (BLK - 1), (L,))
                col = c + (k * 4 + t) * BLK
                lo = plsc.load_gather(buf, [lanes, col])
                hi = plsc.load_gather(buf, [lanes + L, col])
                cols.append((lo, hi))
            p = (cols[0][0] * cols[1][0] + cols[0][1] * cols[1][1] +
                 cols[2][0] * cols[3][0] + cols[2][1] * cols[3][1])
            s = jnp.sum(p)
            score = jnp.where(lanes == lane, s, score)
        return score

    AHEAD = NSET - 1

    def group_body(g, carry):
        vecs = load_vecs(g)
        vecs_next = load_vecs(jnp.minimum(g + 1, NG - 1))
        score = jnp.zeros((L,), jnp.float32)
        for w2 in range(WPG):
            tgt = w2 + AHEAD
            if tgt < WPG:
                fire(vecs, tgt, tgt % NSET)
            else:
                @pl.when(g + 1 < NG)
                def _():
                    fire(vecs_next, tgt - WPG, (tgt - WPG) % NSET)
            drain(w2 % NSET)
            score = compute(vecs, w2, w2 % NSET, score)
        out_v[pl.ds(g * L, L)] = 1.0 / (1.0 + jnp.exp(-score))
        return carry

    first = load_vecs(0)
    for w2 in range(AHEAD):
        fire(first, w2, w2 % NSET)
    lax.fori_loop(0, NG, group_body, 0)
    pltpu.sync_copy(out_v, out_hbm.at[pl.ds(base, BPW)])


@jax.jit
def _fpmc(uid, lic, nit, UIt, IUt, LIt, ILt):
    fn = pl.kernel(
        _fpmc_body,
        out_type=jax.ShapeDtypeStruct((B,), jnp.float32),
        mesh=plsc.VectorSubcoreMesh(core_axis_name="c", subcore_axis_name="s",
                                    num_cores=NC, num_subcores=NS),
        scratch_types=[
            pltpu.VMEM((BPW,), jnp.int32),
            pltpu.VMEM((BPW,), jnp.int32),
            pltpu.VMEM((BPW,), jnp.int32),
            pltpu.VMEM((D, SLOT), jnp.float32),
            pltpu.VMEM((D, SLOT), jnp.float32),
            pltpu.VMEM((D, SLOT), jnp.float32),
            pltpu.VMEM((D, SLOT), jnp.float32),
            pltpu.VMEM((BPW,), jnp.float32),
            pltpu.SemaphoreType.DMA,
            pltpu.SemaphoreType.DMA,
            pltpu.SemaphoreType.DMA,
            pltpu.SemaphoreType.DMA,
        ],
        compiler_params=pltpu.CompilerParams(use_tc_tiling_on_sc=True,
                                             needs_layout_passes=False),
    )
    return fn(uid, lic, nit, UIt, IUt, LIt, ILt)


def kernel(user_id, item_last_click, next_item, UI, IU, LI, IL):
    uid = user_id.reshape(-1).astype(jnp.int32)
    lic = item_last_click.reshape(-1).astype(jnp.int32)
    nit = next_item.reshape(-1).astype(jnp.int32)
    return _fpmc(uid, lic, nit, UI.T, IU.T, LI.T, IL.T)


# speedup vs baseline: 4.3108x; 1.0742x over previous
import jax
import jax.numpy as jnp
from jax import lax
from jax.experimental import pallas as pl
from jax.experimental.pallas import tpu as pltpu
from jax.experimental.pallas import tpu_sc as plsc

B = 16384
D = 32
NC = 2
NS = 16
L = 16
NW = NC * NS
BPW = B // NW
NG = BPW // L
WPG = 2 * L
NSET = 8
BLK = 128
SLOT = 2 * BLK


def _fpmc_body(uid_hbm, lic_hbm, nit_hbm, ui_hbm, iu_hbm, li_hbm, il_hbm,
               out_hbm, idx_u, idx_l, idx_n, blk0, blk1, blk2, blk3, blk4,
               blk5, blk6, blk7, out_v, sem0, sem1, sem2, sem3, sem4, sem5,
               sem6, sem7):
    wid = lax.axis_index("s") * NC + lax.axis_index("c")
    base = wid * BPW

    pltpu.sync_copy(uid_hbm.at[pl.ds(base, BPW)], idx_u)
    pltpu.sync_copy(lic_hbm.at[pl.ds(base, BPW)], idx_l)
    pltpu.sync_copy(nit_hbm.at[pl.ds(base, BPW)], idx_n)

    lanes = lax.iota(jnp.int32, L)
    tabs = (ui_hbm, iu_hbm, li_hbm, il_hbm)
    bufs = (blk0, blk1, blk2, blk3, blk4, blk5, blk6, blk7)
    sems = (sem0, sem1, sem2, sem3, sem4, sem5, sem6, sem7)

    def load_vecs(g):
        s = pl.ds(g * L, L)
        return idx_u[s], idx_l[s], idx_n[s]

    def fire(vecs, w2, pb):
        uvec, lvec, nvec = vecs
        buf, sem = bufs[pb], sems[pb]
        lane = w2 // 2
        if w2 % 2 == 0:
            vs = (uvec[lane], nvec[lane])
            tt = (tabs[0], tabs[1])
        else:
            vs = (lvec[lane], nvec[lane])
            tt = (tabs[2], tabs[3])
        for t in range(2):
            j = pl.multiple_of((vs[t] >> 7) * BLK, BLK)
            pltpu.async_copy(tt[t].at[:, pl.ds(j, BLK)],
                             buf.at[:, pl.ds(t * BLK, BLK)], sem)

    def drain(pb):
        buf, sem = bufs[pb], sems[pb]
        for i in range(2):
            pltpu.make_async_copy(ui_hbm.at[:, pl.ds(0, BLK)],
                                  buf.at[:, pl.ds(i * BLK, BLK)], sem).wait()

    def compute(vecs, w2, score):
        uvec, lvec, nvec = vecs
        lane = w2 // 2
        vs = (uvec[lane], nvec[lane], lvec[lane], nvec[lane])
        bsel = (bufs[(w2 - 1) % NSET], bufs[(w2 - 1) % NSET],
                bufs[w2 % NSET], bufs[w2 % NSET])
        cols = []
        for t in range(4):
            c = jnp.broadcast_to(vs[t] & (BLK - 1), (L,))
            col = c + (t % 2) * BLK
            lo = plsc.load_gather(bsel[t], [lanes, col])
            hi = plsc.load_gather(bsel[t], [lanes + L, col])
            cols.append((lo, hi))
        p = (cols[0][0] * cols[1][0] + cols[0][1] * cols[1][1] +
             cols[2][0] * cols[3][0] + cols[2][1] * cols[3][1])
        s = jnp.sum(p)
        return jnp.where(lanes == lane, s, score)

    AHEAD = NSET - 1

    def group_body(g, carry):
        vecs = load_vecs(g)
        vecs_next = load_vecs(jnp.minimum(g + 1, NG - 1))
        score = jnp.zeros((L,), jnp.float32)
        for w2 in range(WPG):
            tgt = w2 + AHEAD
            if tgt < WPG:
                fire(vecs, tgt, tgt % NSET)
            else:
                @pl.when(g + 1 < NG)
                def _():
                    fire(vecs_next, tgt - WPG, (tgt - WPG) % NSET)
            drain(w2 % NSET)
            if w2 % 2 == 1:
                score = compute(vecs, w2, score)
        out_v[pl.ds(g * L, L)] = 1.0 / (1.0 + jnp.exp(-score))
        return carry

    first = load_vecs(0)
    for w2 in range(AHEAD):
        fire(first, w2, w2 % NSET)
    lax.fori_loop(0, NG, group_body, 0)
    pltpu.sync_copy(out_v, out_hbm.at[pl.ds(base, BPW)])


@jax.jit
def _fpmc(uid, lic, nit, UIt, IUt, LIt, ILt):
    fn = pl.kernel(
        _fpmc_body,
        out_type=jax.ShapeDtypeStruct((B,), jnp.float32),
        mesh=plsc.VectorSubcoreMesh(core_axis_name="c", subcore_axis_name="s",
                                    num_cores=NC, num_subcores=NS),
        scratch_types=[
            pltpu.VMEM((BPW,), jnp.int32),
            pltpu.VMEM((BPW,), jnp.int32),
            pltpu.VMEM((BPW,), jnp.int32),
            pltpu.VMEM((D, SLOT), jnp.float32),
            pltpu.VMEM((D, SLOT), jnp.float32),
            pltpu.VMEM((D, SLOT), jnp.float32),
            pltpu.VMEM((D, SLOT), jnp.float32),
            pltpu.VMEM((D, SLOT), jnp.float32),
            pltpu.VMEM((D, SLOT), jnp.float32),
            pltpu.VMEM((D, SLOT), jnp.float32),
            pltpu.VMEM((D, SLOT), jnp.float32),
            pltpu.VMEM((BPW,), jnp.float32),
            pltpu.SemaphoreType.DMA,
            pltpu.SemaphoreType.DMA,
            pltpu.SemaphoreType.DMA,
            pltpu.SemaphoreType.DMA,
            pltpu.SemaphoreType.DMA,
            pltpu.SemaphoreType.DMA,
            pltpu.SemaphoreType.DMA,
            pltpu.SemaphoreType.DMA,
        ],
        compiler_params=pltpu.CompilerParams(use_tc_tiling_on_sc=True,
                                             needs_layout_passes=False),
    )
    return fn(uid, lic, nit, UIt, IUt, LIt, ILt)


def kernel(user_id, item_last_click, next_item, UI, IU, LI, IL):
    uid = user_id.reshape(-1).astype(jnp.int32)
    lic = item_last_click.reshape(-1).astype(jnp.int32)
    nit = next_item.reshape(-1).astype(jnp.int32)
    return _fpmc(uid, lic, nit, UI.T, IU.T, LI.T, IL.T)
